# baseline (speedup 1.0000x reference)
import functools

import jax
import jax.numpy as jnp
from jax import lax
from jax.experimental import pallas as pl
from jax.experimental.pallas import tpu as pltpu
from jax.experimental.pallas import tpu_sc as plsc

_N, _K, _H, _EF, _A = 10000, 16, 128, 240, 20
_B = 400
_BK = _B * _K
_W = 128
_GA, _GB = 13, 12
_NA, _NB = _GA * _B, _GB * _B
_MA, _MB = _NA * _K, _NB * _K
_MAP = 86016
_MBP = 81920


def _sc_gather(table, idx2d):
    m = idx2d.shape[1]
    h = table.shape[1]
    nw = 32
    win = _W * _H // h
    wpw = m // (win * nw)
    chunk = wpw * win
    nbuf = 6
    mesh = plsc.VectorSubcoreMesh(core_axis_name="core", subcore_axis_name="subcore")

    @functools.partial(
        pl.kernel,
        out_type=jax.ShapeDtypeStruct((m, h), table.dtype),
        mesh=mesh,
        scratch_types=[
            pltpu.VMEM((chunk,), jnp.int32),
            pltpu.VMEM((nbuf, win, h), table.dtype),
            pltpu.SemaphoreType.DMA((nbuf,)),
            pltpu.SemaphoreType.DMA((nbuf,)),
        ],
    )
    def gk(x_hbm, i_hbm, o_hbm, idx_v, bufs, sg, so):
        wid = lax.axis_index("subcore") * 2 + lax.axis_index("core")
        base = wid * chunk
        pltpu.sync_copy(i_hbm.at[0, pl.ds(base, chunk)], idx_v)
        gd, od = {}, {}
        for w in range(min(nbuf, wpw)):
            gd[w] = pltpu.async_copy(
                x_hbm.at[idx_v.at[pl.ds(w * win, win)]], bufs.at[w], sg.at[w])
        for w in range(wpw):
            b = w % nbuf
            gd[w].wait()
            od[w] = pltpu.async_copy(
                bufs.at[b], o_hbm.at[pl.ds(base + w * win, win)], so.at[b])
            nxt = w + nbuf
            if nxt < wpw:
                od[w].wait()
                gd[nxt] = pltpu.async_copy(
                    x_hbm.at[idx_v.at[pl.ds(nxt * win, win)]], bufs.at[b], sg.at[b])
        for w in range(max(0, wpw - nbuf), wpw):
            od[w].wait()

    return gk(table, idx2d)


def _dot(a, b):
    return jnp.dot(a, b, preferred_element_type=jnp.float32)


def _expand(s):
    return jnp.broadcast_to(s[:, None, :], (_B, _K, _H)).reshape(_BK, _H)


def _ksum(x):
    return x.reshape(_B, _K, _H).sum(axis=1)


def _ln(x, s, b):
    m = jnp.mean(x, axis=-1, keepdims=True)
    v = jnp.mean((x - m) ** 2, axis=-1, keepdims=True)
    return (x - m) * lax.rsqrt(v + 1e-5) * s + b


def _node_update(n, agg, wn1, bn1, wn2, bn2, lns, lnb):
    u = jnp.maximum(_dot(n, wn1[:_H]) + _dot(agg, wn1[_H:]) + bn1, 0.0)
    x = n + _dot(u, wn2) + bn2
    return _ln(x, lns, lnb)


_ROW = lambda i: (i, 0)
_FIX = lambda i: (0, 0)


def _bspec(rows, cols, mapped=True):
    return pl.BlockSpec((rows, cols), _ROW if mapped else _FIX)


def _wspecs(shapes):
    return [_bspec(r, c, mapped=False) for (r, c) in shapes]


def _embed_pre(nf, node_w, node_b, wm1ab, bm1):
    def body(nf_r, w_r, b_r, wab_r, bm_r, n_o, p_o, s_o):
        n = _dot(nf_r[...], w_r[...]) + b_r[...]
        n_o[...] = n
        wab = wab_r[...]
        p_o[...] = _dot(n, wab[_H:])
        s_o[...] = _dot(n, wab[:_H]) + bm_r[...]

    out = jax.ShapeDtypeStruct((_N, _H), jnp.float32)
    return pl.pallas_call(
        body,
        grid=(1,),
        in_specs=[_bspec(_N, nf.shape[1])]
        + _wspecs([(nf.shape[1], _H), (1, _H), (2 * _H, _H), (1, _H)]),
        out_specs=[_bspec(_N, _H)] * 3,
        out_shape=[out, out, out],
    )(nf, node_w, node_b, wm1ab, bm1)


def _msg_node1(half, ef, g, s_m, nodes, edge_w, edge_b, wm1c, wm2, bm2,
               wn1, bn1, wn2, bn2, lns, lnb, we1ab, be1, wm1ab_n, bm1_n):
    grid, off = (_GA, 0) if half == 0 else (_GB, _GA)
    nn, mm = grid * _B, grid * _BK

    def body(ef_r, g_r, s_r, n_r, ew_r, eb_r, wm1c_r, wm2_r, bm2_r,
             wn1_r, bn1_r, wn2_r, bn2_r, lns_r, lnb_r, wab_r, be1_r,
             wabn_r, bmn_r, e_o, n_o, pc_o, se_o, sm_o):
        e = _dot(ef_r[...].reshape(_BK, _EF), ew_r[...]) + eb_r[...]
        e_o[...] = e
        h = jnp.maximum(_expand(s_r[...]) + g_r[...] + _dot(e, wm1c_r[...]), 0.0)
        agg = _dot(_ksum(h), wm2_r[...]) + _K * bm2_r[...]
        newn = _node_update(n_r[...], agg, wn1_r[...], bn1_r[...],
                            wn2_r[...], bn2_r[...], lns_r[...], lnb_r[...])
        n_o[...] = newn
        wab = wab_r[...]
        wabn = wabn_r[...]
        pc_o[...] = jnp.concatenate(
            [_dot(newn, wab[_H:]), _dot(newn, wabn[_H:])], axis=1)
        se_o[...] = _dot(newn, wab[:_H]) + be1_r[...]
        sm_o[...] = _dot(newn, wabn[:_H]) + bmn_r[...]

    nh = jax.ShapeDtypeStruct((nn, _H), jnp.float32)
    nh2 = jax.ShapeDtypeStruct((nn, 2 * _H), jnp.float32)
    return pl.pallas_call(
        body,
        grid=(grid,),
        in_specs=[pl.BlockSpec((_B, _K, _EF), lambda i: (i + off, 0, 0)),
                  _bspec(_BK, _H),
                  pl.BlockSpec((_B, _H), lambda i: (i + off, 0)),
                  pl.BlockSpec((_B, _H), lambda i: (i + off, 0))]
        + _wspecs([(_EF, _H), (1, _H), (_H, _H), (_H, _H), (1, _H),
                   (2 * _H, _H), (1, _H), (_H, _H), (1, _H), (1, _H), (1, _H),
                   (2 * _H, _H), (1, _H), (2 * _H, _H), (1, _H)]),
        out_specs=[_bspec(_BK, _H), _bspec(_B, _H), _bspec(_B, 2 * _H),
                   _bspec(_B, _H), _bspec(_B, _H)],
        out_shape=[jax.ShapeDtypeStruct((mm, _H), jnp.float32),
                   nh, nh2, nh, nh],
    )(ef, g, s_m, nodes, edge_w, edge_b, wm1c, wm2, bm2,
      wn1, bn1, wn2, bn2, lns, lnb, we1ab, be1, wm1ab_n, bm1_n)


def _msg_node2(grid, e, g2, s_m, nodes, wm1c, wm2, bm2,
               wn1, bn1, wn2, bn2, lns, lnb, we1ab, be1, wm1ab_n, bm1_n):
    nn = grid * _B

    def body(e_r, g_r, s_r, n_r, wm1c_r, wm2_r, bm2_r,
             wn1_r, bn1_r, wn2_r, bn2_r, lns_r, lnb_r, wab_r, be1_r,
             wabn_r, bmn_r, n_o, pc_o, se_o, sm_o):
        h = jnp.maximum(_expand(s_r[...]) + g_r[...] + _dot(e_r[...], wm1c_r[...]), 0.0)
        agg = _dot(_ksum(h), wm2_r[...]) + _K * bm2_r[...]
        newn = _node_update(n_r[...], agg, wn1_r[...], bn1_r[...],
                            wn2_r[...], bn2_r[...], lns_r[...], lnb_r[...])
        n_o[...] = newn
        wab = wab_r[...]
        wabn = wabn_r[...]
        pc_o[...] = jnp.concatenate(
            [_dot(newn, wab[_H:]), _dot(newn, wabn[_H:])], axis=1)
        se_o[...] = _dot(newn, wab[:_H]) + be1_r[...]
        sm_o[...] = _dot(newn, wabn[:_H]) + bmn_r[...]

    nh = jax.ShapeDtypeStruct((nn, _H), jnp.float32)
    nh2 = jax.ShapeDtypeStruct((nn, 2 * _H), jnp.float32)
    return pl.pallas_call(
        body,
        grid=(grid,),
        in_specs=[_bspec(_BK, _H),
                  pl.BlockSpec((_BK, _H), lambda i: (i, 1)),
                  _bspec(_B, _H), _bspec(_B, _H)]
        + _wspecs([(_H, _H), (_H, _H), (1, _H),
                   (2 * _H, _H), (1, _H), (_H, _H), (1, _H), (1, _H), (1, _H),
                   (2 * _H, _H), (1, _H), (2 * _H, _H), (1, _H)]),
        out_specs=[_bspec(_B, _H), _bspec(_B, 2 * _H),
                   _bspec(_B, _H), _bspec(_B, _H)],
        out_shape=[nh, nh2, nh, nh],
    )(e, g2, s_m, nodes, wm1c, wm2, bm2, wn1, bn1, wn2, bn2,
      lns, lnb, we1ab, be1, wm1ab_n, bm1_n)


def _msg_node3(grid, e, g2, s_m, nodes, wm1c, wm2, bm2,
               wn1, bn1, wn2, bn2, lns, lnb, head_w, head_b):
    nn = grid * _B

    def body(e_r, g_r, s_r, n_r, wm1c_r, wm2_r, bm2_r,
             wn1_r, bn1_r, wn2_r, bn2_r, lns_r, lnb_r, hw_r, hb_r, l_o):
        h = jnp.maximum(_expand(s_r[...]) + g_r[...] + _dot(e_r[...], wm1c_r[...]), 0.0)
        agg = _dot(_ksum(h), wm2_r[...]) + _K * bm2_r[...]
        newn = _node_update(n_r[...], agg, wn1_r[...], bn1_r[...],
                            wn2_r[...], bn2_r[...], lns_r[...], lnb_r[...])
        l_o[...] = _dot(newn, hw_r[...]) + hb_r[...]

    return pl.pallas_call(
        body,
        grid=(grid,),
        in_specs=[_bspec(_BK, _H),
                  pl.BlockSpec((_BK, _H), lambda i: (i, 1)),
                  _bspec(_B, _H), _bspec(_B, _H)]
        + _wspecs([(_H, _H), (_H, _H), (1, _H),
                   (2 * _H, _H), (1, _H), (_H, _H), (1, _H), (1, _H), (1, _H),
                   (_H, _A), (1, _A)]),
        out_specs=[_bspec(_B, _A)],
        out_shape=[jax.ShapeDtypeStruct((nn, _A), jnp.float32)],
    )(e, g2, s_m, nodes, wm1c, wm2, bm2, wn1, bn1, wn2, bn2,
      lns, lnb, head_w, head_b)[0]


def _edge_phase(grid, e, g2, s_e, we1c, we2, be2, les, leb):
    def body(e_r, g_r, s_r, we1c_r, we2_r, be2_r, les_r, leb_r, e_o):
        e = e_r[...]
        h = jnp.maximum(_expand(s_r[...]) + g_r[...] + _dot(e, we1c_r[...]), 0.0)
        x = e + _dot(h, we2_r[...]) + be2_r[...]
        e_o[...] = _ln(x, les_r[...], leb_r[...])

    return pl.pallas_call(
        body,
        grid=(grid,),
        in_specs=[_bspec(_BK, _H),
                  pl.BlockSpec((_BK, _H), lambda i: (i, 0)),
                  _bspec(_B, _H)]
        + _wspecs([(_H, _H), (_H, _H), (1, _H), (1, _H), (1, _H)]),
        out_specs=[_bspec(_BK, _H)],
        out_shape=[jax.ShapeDtypeStruct((grid * _BK, _H), jnp.float32)],
    )(e, g2, s_e, we1c, we2, be2, les, leb)[0]


def kernel(node_features, edge_features, knn_indices,
           node_w, node_b, edge_w, edge_b,
           Wm1, bm1, Wm2, bm2, Wn1, bn1, Wn2, bn2,
           We1, be1, We2, be2, lnn_s, lnn_b, lne_s, lne_b,
           head_w, head_b):
    r1 = lambda v: v.reshape(1, -1)
    knn = knn_indices.astype(jnp.int32).reshape(-1)
    idx_a = jnp.concatenate(
        [knn[:_MA], jnp.zeros((_MAP - _MA,), jnp.int32)]).reshape(1, _MAP)
    idx_b = jnp.concatenate(
        [knn[_MA:], jnp.zeros((_MBP - _MB,), jnp.int32)]).reshape(1, _MBP)

    wm1ab = [Wm1[i][: 2 * _H] for i in range(3)]
    wm1c = [Wm1[i][2 * _H:] for i in range(3)]
    we1ab = [We1[i][: 2 * _H] for i in range(2)]
    we1c = [We1[i][2 * _H:] for i in range(2)]

    nodes0, p1, s1 = _embed_pre(node_features, node_w, r1(node_b),
                                wm1ab[0], r1(bm1[0]))
    g1a = _sc_gather(p1, idx_a)
    g1b = _sc_gather(p1, idx_b)

    def msg1(half, g1h):
        return _msg_node1(
            half, edge_features, g1h, s1, nodes0, edge_w, r1(edge_b),
            wm1c[0], Wm2[0], r1(bm2[0]), Wn1[0], r1(bn1[0]), Wn2[0],
            r1(bn2[0]), r1(lnn_s[0]), r1(lnn_b[0]), we1ab[0], r1(be1[0]),
            wm1ab[1], r1(bm1[1]))

    e1a, n1a, pc1a, se1a, sm2a = msg1(0, g1a)
    e1b, n1b, pc1b, se1b, sm2b = msg1(1, g1b)
    pc1 = jnp.concatenate([pc1a, pc1b], axis=0)
    gc1a = _sc_gather(pc1, idx_a)
    gc1b = _sc_gather(pc1, idx_b)

    e2a = _edge_phase(_GA, e1a, gc1a, se1a, we1c[0], We2[0], r1(be2[0]),
                      r1(lne_s[0]), r1(lne_b[0]))
    e2b = _edge_phase(_GB, e1b, gc1b, se1b, we1c[0], We2[0], r1(be2[0]),
                      r1(lne_s[0]), r1(lne_b[0]))

    def msg2(grid, e2h, gh, smh, nh):
        return _msg_node2(
            grid, e2h, gh, smh, nh, wm1c[1], Wm2[1], r1(bm2[1]),
            Wn1[1], r1(bn1[1]), Wn2[1], r1(bn2[1]),
            r1(lnn_s[1]), r1(lnn_b[1]), we1ab[1], r1(be1[1]),
            wm1ab[2], r1(bm1[2]))

    n2a, pc2a, se2a, sm3a = msg2(_GA, e2a, gc1a, sm2a, n1a)
    n2b, pc2b, se2b, sm3b = msg2(_GB, e2b, gc1b, sm2b, n1b)
    pc2 = jnp.concatenate([pc2a, pc2b], axis=0)
    gc2a = _sc_gather(pc2, idx_a)
    gc2b = _sc_gather(pc2, idx_b)

    e3a = _edge_phase(_GA, e2a, gc2a, se2a, we1c[1], We2[1], r1(be2[1]),
                      r1(lne_s[1]), r1(lne_b[1]))
    e3b = _edge_phase(_GB, e2b, gc2b, se2b, we1c[1], We2[1], r1(be2[1]),
                      r1(lne_s[1]), r1(lne_b[1]))

    def msg3(grid, e3h, gh, smh, nh):
        return _msg_node3(
            grid, e3h, gh, smh, nh, wm1c[2], Wm2[2], r1(bm2[2]),
            Wn1[2], r1(bn1[2]), Wn2[2], r1(bn2[2]),
            r1(lnn_s[2]), r1(lnn_b[2]), head_w, r1(head_b))

    la = msg3(_GA, e3a, gc2a, sm3a, n2a)
    lb = msg3(_GB, e3b, gc2b, sm3b, n2b)
    return jnp.concatenate([la, lb], axis=0)


# speedup vs baseline: 1.4233x; 1.4233x over previous
import functools

import jax
import jax.numpy as jnp
from jax import lax
from jax.experimental import pallas as pl
from jax.experimental.pallas import tpu as pltpu
from jax.experimental.pallas import tpu_sc as plsc

_N, _K, _H, _EF, _A = 10000, 16, 128, 240, 20
_B = 400
_G = _N // _B
_BK = _B * _K
_W = 128
_MPAD = 163840


def _sc_gather(table, idx2d):
    m = idx2d.shape[1]
    h = table.shape[1]
    nw = 32
    win = _W * _H // h
    wpw = m // (win * nw)
    chunk = wpw * win
    nbuf = 6
    mesh = plsc.VectorSubcoreMesh(core_axis_name="core", subcore_axis_name="subcore")

    @functools.partial(
        pl.kernel,
        out_type=jax.ShapeDtypeStruct((m, h), table.dtype),
        mesh=mesh,
        scratch_types=[
            pltpu.VMEM((chunk,), jnp.int32),
            pltpu.VMEM((nbuf, win, h), table.dtype),
            pltpu.SemaphoreType.DMA((nbuf,)),
            pltpu.SemaphoreType.DMA((nbuf,)),
        ],
    )
    def gk(x_hbm, i_hbm, o_hbm, idx_v, bufs, sg, so):
        wid = lax.axis_index("subcore") * 2 + lax.axis_index("core")
        base = wid * chunk
        pltpu.sync_copy(i_hbm.at[0, pl.ds(base, chunk)], idx_v)
        gd, od = {}, {}
        for w in range(min(nbuf, wpw)):
            gd[w] = pltpu.async_copy(
                x_hbm.at[idx_v.at[pl.ds(w * win, win)]], bufs.at[w], sg.at[w])
        for w in range(wpw):
            b = w % nbuf
            gd[w].wait()
            od[w] = pltpu.async_copy(
                bufs.at[b], o_hbm.at[pl.ds(base + w * win, win)], so.at[b])
            nxt = w + nbuf
            if nxt < wpw:
                od[w].wait()
                gd[nxt] = pltpu.async_copy(
                    x_hbm.at[idx_v.at[pl.ds(nxt * win, win)]], bufs.at[b], sg.at[b])
        for w in range(max(0, wpw - nbuf), wpw):
            od[w].wait()

    return gk(table, idx2d)


def _dot(a, b):
    return jnp.dot(a, b, preferred_element_type=jnp.float32)


def _expand(s):
    return jnp.broadcast_to(s[:, None, :], (_B, _K, _H)).reshape(_BK, _H)


def _ksum(x):
    return x.reshape(_B, _K, _H).sum(axis=1)


def _ln(x, s, b):
    m = jnp.mean(x, axis=-1, keepdims=True)
    v = jnp.mean((x - m) ** 2, axis=-1, keepdims=True)
    return (x - m) * lax.rsqrt(v + 1e-5) * s + b


def _node_update(n, agg, wn1, bn1, wn2, bn2, lns, lnb):
    u = jnp.maximum(_dot(n, wn1[:_H]) + _dot(agg, wn1[_H:]) + bn1, 0.0)
    x = n + _dot(u, wn2) + bn2
    return _ln(x, lns, lnb)


_ROW = lambda i: (i, 0)
_FIX = lambda i: (0, 0)


def _bspec(rows, cols, mapped=True):
    return pl.BlockSpec((rows, cols), _ROW if mapped else _FIX)


def _wspecs(shapes):
    return [_bspec(r, c, mapped=False) for (r, c) in shapes]


def _embed_pre(nf, node_w, node_b, wm1ab, bm1):
    def body(nf_r, w_r, b_r, wab_r, bm_r, n_o, p_o, s_o):
        n = _dot(nf_r[...], w_r[...]) + b_r[...]
        n_o[...] = n
        wab = wab_r[...]
        p_o[...] = _dot(n, wab[_H:])
        s_o[...] = _dot(n, wab[:_H]) + bm_r[...]

    out = jax.ShapeDtypeStruct((_N, _H), jnp.float32)
    return pl.pallas_call(
        body,
        grid=(1,),
        in_specs=[_bspec(_N, nf.shape[1])]
        + _wspecs([(nf.shape[1], _H), (1, _H), (2 * _H, _H), (1, _H)]),
        out_specs=[_bspec(_N, _H)] * 3,
        out_shape=[out, out, out],
    )(nf, node_w, node_b, wm1ab, bm1)


def _msg_node1(ef, g, s_m, nodes, edge_w, edge_b, wm1c, wm2, bm2,
               wn1, bn1, wn2, bn2, lns, lnb, we1ab, be1, wm1ab_n, bm1_n):
    def body(ef_r, g_r, s_r, n_r, ew_r, eb_r, wm1c_r, wm2_r, bm2_r,
             wn1_r, bn1_r, wn2_r, bn2_r, lns_r, lnb_r, wab_r, be1_r,
             wabn_r, bmn_r, e_o, n_o, pc_o, se_o, sm_o):
        e = _dot(ef_r[...].reshape(_BK, _EF), ew_r[...]) + eb_r[...]
        e_o[...] = e
        h = jnp.maximum(_expand(s_r[...]) + g_r[...] + _dot(e, wm1c_r[...]), 0.0)
        agg = _dot(_ksum(h), wm2_r[...]) + _K * bm2_r[...]
        newn = _node_update(n_r[...], agg, wn1_r[...], bn1_r[...],
                            wn2_r[...], bn2_r[...], lns_r[...], lnb_r[...])
        n_o[...] = newn
        wab = wab_r[...]
        wabn = wabn_r[...]
        pc_o[...] = jnp.concatenate(
            [_dot(newn, wab[_H:]), _dot(newn, wabn[_H:])], axis=1)
        se_o[...] = _dot(newn, wab[:_H]) + be1_r[...]
        sm_o[...] = _dot(newn, wabn[:_H]) + bmn_r[...]

    nh = jax.ShapeDtypeStruct((_N, _H), jnp.float32)
    nh2 = jax.ShapeDtypeStruct((_N, 2 * _H), jnp.float32)
    return pl.pallas_call(
        body,
        grid=(_G,),
        in_specs=[pl.BlockSpec((_B, _K, _EF), lambda i: (i, 0, 0)),
                  _bspec(_BK, _H), _bspec(_B, _H), _bspec(_B, _H)]
        + _wspecs([(_EF, _H), (1, _H), (_H, _H), (_H, _H), (1, _H),
                   (2 * _H, _H), (1, _H), (_H, _H), (1, _H), (1, _H), (1, _H),
                   (2 * _H, _H), (1, _H), (2 * _H, _H), (1, _H)]),
        out_specs=[_bspec(_BK, _H), _bspec(_B, _H), _bspec(_B, 2 * _H),
                   _bspec(_B, _H), _bspec(_B, _H)],
        out_shape=[jax.ShapeDtypeStruct((_N * _K, _H), jnp.float32),
                   nh, nh2, nh, nh],
    )(ef, g, s_m, nodes, edge_w, edge_b, wm1c, wm2, bm2,
      wn1, bn1, wn2, bn2, lns, lnb, we1ab, be1, wm1ab_n, bm1_n)


def _edge_msg_node(e, gc, s_e, s_m, nodes,
                   we1c, we2, be2, les, leb,
                   wm1c, wm2, bm2, wn1, bn1, wn2, bn2, lns, lnb,
                   we1ab, be1, wm1ab_n, bm1_n, write_e):
    def body(e_r, gc_r, se_r, sm_r, n_r, we1c_r, we2_r, be2_r, les_r, leb_r,
             wm1c_r, wm2_r, bm2_r, wn1_r, bn1_r, wn2_r, bn2_r, lns_r, lnb_r,
             wab_r, be1_r, wabn_r, bmn_r, *outs):
        if write_e:
            e_o, n_o, pc_o, se_o, sm_o = outs
        else:
            n_o, pc_o, se_o, sm_o = outs
        e = e_r[...]
        gc = gc_r[...]
        he = jnp.maximum(_expand(se_r[...]) + gc[:, :_H] + _dot(e, we1c_r[...]), 0.0)
        e_new = _ln(e + _dot(he, we2_r[...]) + be2_r[...], les_r[...], leb_r[...])
        if write_e:
            e_o[...] = e_new
        h = jnp.maximum(_expand(sm_r[...]) + gc[:, _H:] + _dot(e_new, wm1c_r[...]), 0.0)
        agg = _dot(_ksum(h), wm2_r[...]) + _K * bm2_r[...]
        newn = _node_update(n_r[...], agg, wn1_r[...], bn1_r[...],
                            wn2_r[...], bn2_r[...], lns_r[...], lnb_r[...])
        n_o[...] = newn
        wab = wab_r[...]
        wabn = wabn_r[...]
        pc_o[...] = jnp.concatenate(
            [_dot(newn, wab[_H:]), _dot(newn, wabn[_H:])], axis=1)
        se_o[...] = _dot(newn, wab[:_H]) + be1_r[...]
        sm_o[...] = _dot(newn, wabn[:_H]) + bmn_r[...]

    nh = jax.ShapeDtypeStruct((_N, _H), jnp.float32)
    nh2 = jax.ShapeDtypeStruct((_N, 2 * _H), jnp.float32)
    out_specs = [_bspec(_B, _H), _bspec(_B, 2 * _H), _bspec(_B, _H),
                 _bspec(_B, _H)]
    out_shape = [nh, nh2, nh, nh]
    if write_e:
        out_specs = [_bspec(_BK, _H)] + out_specs
        out_shape = [jax.ShapeDtypeStruct((_N * _K, _H), jnp.float32)] + out_shape
    return pl.pallas_call(
        body,
        grid=(_G,),
        in_specs=[_bspec(_BK, _H), _bspec(_BK, 2 * _H), _bspec(_B, _H),
                  _bspec(_B, _H), _bspec(_B, _H)]
        + _wspecs([(_H, _H), (_H, _H), (1, _H), (1, _H), (1, _H),
                   (_H, _H), (_H, _H), (1, _H),
                   (2 * _H, _H), (1, _H), (_H, _H), (1, _H), (1, _H), (1, _H),
                   (2 * _H, _H), (1, _H), (2 * _H, _H), (1, _H)]),
        out_specs=out_specs,
        out_shape=out_shape,
    )(e, gc, s_e, s_m, nodes, we1c, we2, be2, les, leb,
      wm1c, wm2, bm2, wn1, bn1, wn2, bn2, lns, lnb,
      we1ab, be1, wm1ab_n, bm1_n)


def _edge_msg_head(e, gc, s_e, s_m, nodes,
                   we1c, we2, be2, les, leb,
                   wm1c, wm2, bm2, wn1, bn1, wn2, bn2, lns, lnb,
                   head_w, head_b):
    def body(e_r, gc_r, se_r, sm_r, n_r, we1c_r, we2_r, be2_r, les_r, leb_r,
             wm1c_r, wm2_r, bm2_r, wn1_r, bn1_r, wn2_r, bn2_r, lns_r, lnb_r,
             hw_r, hb_r, l_o):
        e = e_r[...]
        gc = gc_r[...]
        he = jnp.maximum(_expand(se_r[...]) + gc[:, :_H] + _dot(e, we1c_r[...]), 0.0)
        e_new = _ln(e + _dot(he, we2_r[...]) + be2_r[...], les_r[...], leb_r[...])
        h = jnp.maximum(_expand(sm_r[...]) + gc[:, _H:] + _dot(e_new, wm1c_r[...]), 0.0)
        agg = _dot(_ksum(h), wm2_r[...]) + _K * bm2_r[...]
        newn = _node_update(n_r[...], agg, wn1_r[...], bn1_r[...],
                            wn2_r[...], bn2_r[...], lns_r[...], lnb_r[...])
        l_o[...] = _dot(newn, hw_r[...]) + hb_r[...]

    return pl.pallas_call(
        body,
        grid=(_G,),
        in_specs=[_bspec(_BK, _H), _bspec(_BK, 2 * _H), _bspec(_B, _H),
                  _bspec(_B, _H), _bspec(_B, _H)]
        + _wspecs([(_H, _H), (_H, _H), (1, _H), (1, _H), (1, _H),
                   (_H, _H), (_H, _H), (1, _H),
                   (2 * _H, _H), (1, _H), (_H, _H), (1, _H), (1, _H), (1, _H),
                   (_H, _A), (1, _A)]),
        out_specs=[_bspec(_B, _A)],
        out_shape=[jax.ShapeDtypeStruct((_N, _A), jnp.float32)],
    )(e, gc, s_e, s_m, nodes, we1c, we2, be2, les, leb,
      wm1c, wm2, bm2, wn1, bn1, wn2, bn2, lns, lnb, head_w, head_b)[0]


def kernel(node_features, edge_features, knn_indices,
           node_w, node_b, edge_w, edge_b,
           Wm1, bm1, Wm2, bm2, Wn1, bn1, Wn2, bn2,
           We1, be1, We2, be2, lnn_s, lnn_b, lne_s, lne_b,
           head_w, head_b):
    r1 = lambda v: v.reshape(1, -1)
    knn = knn_indices.astype(jnp.int32).reshape(-1)
    idx = jnp.concatenate(
        [knn, jnp.zeros((_MPAD - _N * _K,), jnp.int32)]).reshape(1, _MPAD)

    wm1ab = [Wm1[i][: 2 * _H] for i in range(3)]
    wm1c = [Wm1[i][2 * _H:] for i in range(3)]
    we1ab = [We1[i][: 2 * _H] for i in range(2)]
    we1c = [We1[i][2 * _H:] for i in range(2)]

    nodes0, p1, s1 = _embed_pre(node_features, node_w, r1(node_b),
                                wm1ab[0], r1(bm1[0]))
    g1 = _sc_gather(p1, idx)
    e1, n1, pc1, se1, sm2 = _msg_node1(
        edge_features, g1, s1, nodes0, edge_w, r1(edge_b), wm1c[0], Wm2[0],
        r1(bm2[0]), Wn1[0], r1(bn1[0]), Wn2[0], r1(bn2[0]),
        r1(lnn_s[0]), r1(lnn_b[0]), we1ab[0], r1(be1[0]), wm1ab[1], r1(bm1[1]))
    gc1 = _sc_gather(pc1, idx)
    e2, n2, pc2, se2, sm3 = _edge_msg_node(
        e1, gc1, se1, sm2, n1,
        we1c[0], We2[0], r1(be2[0]), r1(lne_s[0]), r1(lne_b[0]),
        wm1c[1], Wm2[1], r1(bm2[1]), Wn1[1], r1(bn1[1]), Wn2[1], r1(bn2[1]),
        r1(lnn_s[1]), r1(lnn_b[1]),
        we1ab[1], r1(be1[1]), wm1ab[2], r1(bm1[2]), True)
    gc2 = _sc_gather(pc2, idx)
    logits = _edge_msg_head(
        e2, gc2, se2, sm3, n2,
        we1c[1], We2[1], r1(be2[1]), r1(lne_s[1]), r1(lne_b[1]),
        wm1c[2], Wm2[2], r1(bm2[2]), Wn1[2], r1(bn1[2]), Wn2[2], r1(bn2[2]),
        r1(lnn_s[2]), r1(lnn_b[2]), head_w, r1(head_b))
    return logits
